# baseline (speedup 1.0000x reference)
import functools

import jax
import jax.numpy as jnp
from jax import lax
from jax.experimental import pallas as pl
from jax.experimental.pallas import tpu as pltpu
from jax.experimental.pallas import tpu_sc as plsc

NC = 2
NS = 16
L = 16
NW = NC * NS

E = 500_000
EPAD = 524_288
D = 128
CHUNK = 128
TOT_CHUNKS = EPAD // CHUNK
NGRP = CHUNK // L
NBUF = 2
K0 = 104
K1 = 152
KMAX = max(K0, K1)


def _sc_body(h_drug, h_disease, d_idx, e_idx, out_hbm,
             idx_d_v, idx_e_v, rows_d, rows_e, out_v, *sems):
    cid = lax.axis_index("c")
    sid = lax.axis_index("s")
    my_k = jnp.where(cid == 0, K0, K1)
    base_chunk = jnp.where(cid == 0, sid * K0, NS * K0 + sid * K1)
    base = base_chunk * CHUNK

    pltpu.sync_copy(d_idx.at[pl.ds(base, KMAX * CHUNK)], idx_d_v)
    pltpu.sync_copy(e_idx.at[pl.ds(base, KMAX * CHUNK)], idx_e_v)

    sems_d = sems[:NBUF]
    sems_e = sems[NBUF:]

    def gather_start(g, b):
        idx_sl = idx_d_v.at[pl.ds(g * CHUNK, CHUNK)]
        pltpu.async_copy(h_drug.at[idx_sl], rows_d.at[b], sems_d[b])
        idx_sl_e = idx_e_v.at[pl.ds(g * CHUNK, CHUNK)]
        pltpu.async_copy(h_disease.at[idx_sl_e], rows_e.at[b], sems_e[b])

    def gather_wait(g, b):
        idx_sl = idx_d_v.at[pl.ds(g * CHUNK, CHUNK)]
        pltpu.make_async_copy(h_drug.at[idx_sl], rows_d.at[b], sems_d[b]).wait()
        idx_sl_e = idx_e_v.at[pl.ds(g * CHUNK, CHUNK)]
        pltpu.make_async_copy(h_disease.at[idx_sl_e], rows_e.at[b],
                              sems_e[b]).wait()

    iota16 = lax.iota(jnp.int32, L)

    def compute(g, b):
        rd = rows_d.at[b]
        re = rows_e.at[b]
        for grp in range(NGRP):
            base_e = grp * L

            @pl.loop(0, L, init_carry=jnp.zeros((L,), jnp.float32), unroll=2)
            def edge_loop(i, res):
                e = base_e + i
                acc = jnp.zeros((L,), jnp.float32)
                for k in range(D // (2 * L)):
                    dw = plsc.bitcast(rd[e, pl.ds(k * L, L)], jnp.bfloat16)
                    ew = plsc.bitcast(re[e, pl.ds(k * L, L)], jnp.bfloat16)
                    pd = dw * ew
                    lo, hi = plsc.unpack(pd, format=plsc.PackFormat.INTERLEAVED)
                    acc = acc + lo + hi
                s = jnp.full((L,), jnp.sum(acc))
                return jnp.where(iota16 == i, s, res)

            out_v[pl.ds(g * CHUNK + base_e, L)] = edge_loop

    for b in range(NBUF):
        gather_start(b, b)

    @pl.loop(0, my_k, step=NBUF)
    def ring(gg):
        for b in range(NBUF):
            g = gg + b
            gather_wait(g, b)
            compute(g, b)

            @pl.when(g + NBUF < my_k)
            def _():
                gather_start(g + NBUF, b)

    @pl.when(cid == 0)
    def _():
        pltpu.sync_copy(out_v.at[pl.ds(0, K0 * CHUNK)],
                        out_hbm.at[pl.ds(base, K0 * CHUNK)])

    @pl.when(cid == 1)
    def _():
        pltpu.sync_copy(out_v.at[pl.ds(0, K1 * CHUNK)],
                        out_hbm.at[pl.ds(base, K1 * CHUNK)])


@jax.jit
def _link_predict_sc(h_drug, h_disease, d_idx, e_idx):
    mesh = plsc.VectorSubcoreMesh(core_axis_name="c", subcore_axis_name="s")
    k = functools.partial(
        pl.kernel,
        out_type=jax.ShapeDtypeStruct((EPAD,), jnp.float32),
        mesh=mesh,
        compiler_params=pltpu.CompilerParams(
            needs_layout_passes=False, use_tc_tiling_on_sc=False),
        scratch_types=[
            pltpu.VMEM((KMAX * CHUNK,), jnp.int32),
            pltpu.VMEM((KMAX * CHUNK,), jnp.int32),
            pltpu.VMEM((NBUF, CHUNK, D // 2), jnp.int32),
            pltpu.VMEM((NBUF, CHUNK, D // 2), jnp.int32),
            pltpu.VMEM((KMAX * CHUNK,), jnp.float32),
        ] + [pltpu.SemaphoreType.DMA] * (2 * NBUF),
    )(_sc_body)
    return k(h_drug, h_disease, d_idx, e_idx)


def kernel(h_drug, h_disease, edges):
    h_drug = lax.bitcast_convert_type(
        h_drug.astype(jnp.bfloat16).reshape(10000, D // 2, 2), jnp.int32)
    h_disease = lax.bitcast_convert_type(
        h_disease.astype(jnp.bfloat16).reshape(10000, D // 2, 2), jnp.int32)
    d_idx = edges[:, 0].astype(jnp.int32)
    e_idx = edges[:, 1].astype(jnp.int32)
    pad = EPAD - E
    d_idx = jnp.concatenate([d_idx, jnp.zeros((pad,), jnp.int32)])
    e_idx = jnp.concatenate([e_idx, jnp.zeros((pad,), jnp.int32)])
    out = _link_predict_sc(h_drug, h_disease, d_idx, e_idx)
    return out[:E]


# speedup vs baseline: 1.0098x; 1.0098x over previous
import functools

import jax
import jax.numpy as jnp
from jax import lax
from jax.experimental import pallas as pl
from jax.experimental.pallas import tpu as pltpu
from jax.experimental.pallas import tpu_sc as plsc

NC = 2
NS = 16
L = 16
NW = NC * NS

E = 500_000
EPAD = 524_288
D = 128
CHUNK = 128
TOT_CHUNKS = EPAD // CHUNK
NGRP = CHUNK // L
NBUF = 2
K0 = 152
K1 = 104
KMAX = max(K0, K1)


def _sc_body(h_drug, h_disease, d_idx, e_idx, out_hbm,
             idx_d_v, idx_e_v, rows_d, rows_e, out_v, *sems):
    cid = lax.axis_index("c")
    sid = lax.axis_index("s")
    my_k = jnp.where(cid == 0, K0, K1)
    base_chunk = jnp.where(cid == 0, sid * K0, NS * K0 + sid * K1)
    base = base_chunk * CHUNK

    pltpu.sync_copy(d_idx.at[pl.ds(base, KMAX * CHUNK)], idx_d_v)
    pltpu.sync_copy(e_idx.at[pl.ds(base, KMAX * CHUNK)], idx_e_v)

    sems_d = sems[:NBUF]
    sems_e = sems[NBUF:]

    def gather_start(g, b):
        idx_sl = idx_d_v.at[pl.ds(g * CHUNK, CHUNK)]
        pltpu.async_copy(h_drug.at[idx_sl], rows_d.at[b], sems_d[b])
        idx_sl_e = idx_e_v.at[pl.ds(g * CHUNK, CHUNK)]
        pltpu.async_copy(h_disease.at[idx_sl_e], rows_e.at[b], sems_e[b])

    def gather_wait(g, b):
        idx_sl = idx_d_v.at[pl.ds(g * CHUNK, CHUNK)]
        pltpu.make_async_copy(h_drug.at[idx_sl], rows_d.at[b], sems_d[b]).wait()
        idx_sl_e = idx_e_v.at[pl.ds(g * CHUNK, CHUNK)]
        pltpu.make_async_copy(h_disease.at[idx_sl_e], rows_e.at[b],
                              sems_e[b]).wait()

    iota16 = lax.iota(jnp.int32, L)

    def compute(g, b):
        rd = rows_d.at[b]
        re = rows_e.at[b]
        for grp in range(NGRP):
            base_e = grp * L

            @pl.loop(0, L, init_carry=jnp.zeros((L,), jnp.float32), unroll=2)
            def edge_loop(i, res):
                e = base_e + i
                acc = jnp.zeros((L,), jnp.float32)
                for k in range(D // (2 * L)):
                    dw = plsc.bitcast(rd[e, pl.ds(k * L, L)], jnp.bfloat16)
                    ew = plsc.bitcast(re[e, pl.ds(k * L, L)], jnp.bfloat16)
                    pd = dw * ew
                    lo, hi = plsc.unpack(pd, format=plsc.PackFormat.INTERLEAVED)
                    acc = acc + lo + hi
                s = jnp.full((L,), jnp.sum(acc))
                return jnp.where(iota16 == i, s, res)

            out_v[pl.ds(g * CHUNK + base_e, L)] = edge_loop

    for b in range(NBUF):
        gather_start(b, b)

    @pl.loop(0, my_k, step=NBUF)
    def ring(gg):
        for b in range(NBUF):
            g = gg + b
            gather_wait(g, b)
            compute(g, b)

            @pl.when(g + NBUF < my_k)
            def _():
                gather_start(g + NBUF, b)

    @pl.when(cid == 0)
    def _():
        pltpu.sync_copy(out_v.at[pl.ds(0, K0 * CHUNK)],
                        out_hbm.at[pl.ds(base, K0 * CHUNK)])

    @pl.when(cid == 1)
    def _():
        pltpu.sync_copy(out_v.at[pl.ds(0, K1 * CHUNK)],
                        out_hbm.at[pl.ds(base, K1 * CHUNK)])


@jax.jit
def _link_predict_sc(h_drug, h_disease, d_idx, e_idx):
    mesh = plsc.VectorSubcoreMesh(core_axis_name="c", subcore_axis_name="s")
    k = functools.partial(
        pl.kernel,
        out_type=jax.ShapeDtypeStruct((EPAD,), jnp.float32),
        mesh=mesh,
        compiler_params=pltpu.CompilerParams(
            needs_layout_passes=False, use_tc_tiling_on_sc=False),
        scratch_types=[
            pltpu.VMEM((KMAX * CHUNK,), jnp.int32),
            pltpu.VMEM((KMAX * CHUNK,), jnp.int32),
            pltpu.VMEM((NBUF, CHUNK, D // 2), jnp.int32),
            pltpu.VMEM((NBUF, CHUNK, D // 2), jnp.int32),
            pltpu.VMEM((KMAX * CHUNK,), jnp.float32),
        ] + [pltpu.SemaphoreType.DMA] * (2 * NBUF),
    )(_sc_body)
    return k(h_drug, h_disease, d_idx, e_idx)


def kernel(h_drug, h_disease, edges):
    h_drug = lax.bitcast_convert_type(
        h_drug.astype(jnp.bfloat16).reshape(10000, D // 2, 2), jnp.int32)
    h_disease = lax.bitcast_convert_type(
        h_disease.astype(jnp.bfloat16).reshape(10000, D // 2, 2), jnp.int32)
    d_idx = edges[:, 0].astype(jnp.int32)
    e_idx = edges[:, 1].astype(jnp.int32)
    pad = EPAD - E
    d_idx = jnp.concatenate([d_idx, jnp.zeros((pad,), jnp.int32)])
    e_idx = jnp.concatenate([e_idx, jnp.zeros((pad,), jnp.int32)])
    out = _link_predict_sc(h_drug, h_disease, d_idx, e_idx)
    return out[:E]


# speedup vs baseline: 1.1178x; 1.1070x over previous
import functools

import jax
import jax.numpy as jnp
from jax import lax
from jax.experimental import pallas as pl
from jax.experimental.pallas import tpu as pltpu
from jax.experimental.pallas import tpu_sc as plsc

NC = 2
NS = 16
L = 16
NW = NC * NS

E = 500_000
EPAD = 524_288
D = 128
PER_TILE = EPAD // NW
CHUNK = 128
NCHUNK = PER_TILE // CHUNK
NGRP = CHUNK // L
NBUF = 2


def _sc_body(h_drug, h_disease, d_idx, e_idx, out_hbm,
             idx_d_v, idx_e_v, rows_d, rows_e, out_v, *sems):
    wid = lax.axis_index("s") * NC + lax.axis_index("c")
    base = wid * PER_TILE

    pltpu.sync_copy(d_idx.at[pl.ds(base, PER_TILE)], idx_d_v)
    pltpu.sync_copy(e_idx.at[pl.ds(base, PER_TILE)], idx_e_v)

    sems_d = sems[:NBUF]
    sems_e = sems[NBUF:]

    def gather_start(g, b):
        idx_sl = idx_d_v.at[pl.ds(g * CHUNK, CHUNK)]
        pltpu.async_copy(h_drug.at[idx_sl], rows_d.at[b], sems_d[b])
        idx_sl_e = idx_e_v.at[pl.ds(g * CHUNK, CHUNK)]
        pltpu.async_copy(h_disease.at[idx_sl_e], rows_e.at[b], sems_e[b])

    def gather_wait(g, b):
        idx_sl = idx_d_v.at[pl.ds(g * CHUNK, CHUNK)]
        pltpu.make_async_copy(h_drug.at[idx_sl], rows_d.at[b], sems_d[b]).wait()
        idx_sl_e = idx_e_v.at[pl.ds(g * CHUNK, CHUNK)]
        pltpu.make_async_copy(h_disease.at[idx_sl_e], rows_e.at[b],
                              sems_e[b]).wait()

    iota16 = lax.iota(jnp.int32, L)

    def compute(g, b):
        rd = rows_d.at[b]
        re = rows_e.at[b]
        for grp in range(NGRP):
            base_e = grp * L

            @pl.loop(0, L, init_carry=jnp.zeros((L,), jnp.float32), unroll=2)
            def edge_loop(i, res):
                e = base_e + i
                acc = jnp.zeros((L,), jnp.float32)
                for k in range(D // (2 * L)):
                    dw = plsc.bitcast(rd[e, pl.ds(k * L, L)], jnp.bfloat16)
                    ew = plsc.bitcast(re[e, pl.ds(k * L, L)], jnp.bfloat16)
                    pd = dw * ew
                    lo, hi = plsc.unpack(pd, format=plsc.PackFormat.INTERLEAVED)
                    acc = acc + lo + hi
                s = jnp.full((L,), jnp.sum(acc))
                return jnp.where(iota16 == i, s, res)

            out_v[pl.ds(g * CHUNK + base_e, L)] = edge_loop

    for b in range(NBUF):
        gather_start(b, b)

    @pl.loop(0, NCHUNK, step=NBUF)
    def ring(gg):
        for b in range(NBUF):
            g = gg + b
            gather_wait(g, b)
            compute(g, b)

            @pl.when(g + NBUF < NCHUNK)
            def _():
                gather_start(g + NBUF, b)

    pltpu.sync_copy(out_v, out_hbm.at[pl.ds(base, PER_TILE)])


@jax.jit
def _link_predict_sc(h_drug, h_disease, d_idx, e_idx):
    mesh = plsc.VectorSubcoreMesh(core_axis_name="c", subcore_axis_name="s")
    k = functools.partial(
        pl.kernel,
        out_type=jax.ShapeDtypeStruct((EPAD,), jnp.float32),
        mesh=mesh,
        compiler_params=pltpu.CompilerParams(
            needs_layout_passes=False, use_tc_tiling_on_sc=False),
        scratch_types=[
            pltpu.VMEM((PER_TILE,), jnp.int32),
            pltpu.VMEM((PER_TILE,), jnp.int32),
            pltpu.VMEM((NBUF, CHUNK, D // 2), jnp.int32),
            pltpu.VMEM((NBUF, CHUNK, D // 2), jnp.int32),
            pltpu.VMEM((PER_TILE,), jnp.float32),
        ] + [pltpu.SemaphoreType.DMA] * (2 * NBUF),
    )(_sc_body)
    return k(h_drug, h_disease, d_idx, e_idx)


def kernel(h_drug, h_disease, edges):
    h_drug = lax.bitcast_convert_type(
        h_drug.astype(jnp.bfloat16).reshape(10000, D // 2, 2), jnp.int32)
    h_disease = lax.bitcast_convert_type(
        h_disease.astype(jnp.bfloat16).reshape(10000, D // 2, 2), jnp.int32)
    d_idx = edges[:, 0].astype(jnp.int32)
    e_idx = edges[:, 1].astype(jnp.int32)
    pad = EPAD - E
    d_idx = jnp.concatenate([d_idx, jnp.zeros((pad,), jnp.int32)])
    e_idx = jnp.concatenate([e_idx, jnp.zeros((pad,), jnp.int32)])
    out = _link_predict_sc(h_drug, h_disease, d_idx, e_idx)
    return out[:E]
